# Initial kernel scaffold; baseline (speedup 1.0000x reference)
import jax
import jax.numpy as jnp
from jax.experimental import pallas as pl


def kernel(x, edge_index, batch, W1, b1, g1, bt1, W2, b2, g2, bt2, W3, b3, g3, bt3, lw, lb):
    raise NotImplementedError("write your pallas kernel here")


# speedup vs baseline: 13.3559x; 13.3559x over previous
import functools

import jax
import jax.numpy as jnp
from jax import lax
from jax.experimental import pallas as pl
from jax.experimental.pallas import tpu as pltpu
from jax.experimental.pallas import tpu_sc as plsc

N = 10000
NP = 10240
E = 320000
D = 128
G = 16
BLK = 1024
NBLK = NP // BLK
CH = 128
NCH = E // CH
NW = 32
RPT = NP // 16

@functools.lru_cache(maxsize=1)
def _mesh():
    return plsc.VectorSubcoreMesh(core_axis_name="c", subcore_axis_name="s")


def _chunk_range(wid):
    base = NCH // NW
    rem = NCH % NW
    lo = wid * base + jnp.minimum(wid, rem)
    cnt = base + jnp.where(wid < rem, 1, 0)
    return lo, cnt


def _zero_rows(rows_v, nrows):
    def body(r, _):
        for j in range(8):
            rows_v[r, pl.ds(j * 16, 16)] = jnp.zeros((16,), jnp.float32)
        return 0
    lax.fori_loop(0, nrows, body, 0)


def _deg_body(dst_hbm, out_hbm, dst_v, ones_v, zrow_v, acc_sh, sem):
    cid = lax.axis_index("c")
    sid = lax.axis_index("s")
    wid = sid * 2 + cid

    for j in range(8):
        ones_v[pl.ds(j * 16, 16)] = jnp.ones((16,), jnp.float32)
    def zb(i, _):
        zrow_v[pl.ds(i * 16, 16)] = jnp.zeros((16,), jnp.float32)
        return 0
    lax.fori_loop(0, RPT // 16, zb, 0)
    pltpu.sync_copy(zrow_v, acc_sh.at[pl.ds(sid * RPT, RPT)])
    plsc.subcore_barrier()

    lo, cnt = _chunk_range(wid)

    def chunk(c, _):
        base = (lo + c) * CH
        pltpu.sync_copy(dst_hbm.at[pl.ds(base, CH)], dst_v)
        pltpu.sync_copy(ones_v, acc_sh.at[dst_v], add=True)
        return 0
    lax.fori_loop(0, cnt, chunk, 0)
    plsc.subcore_barrier()

    pltpu.sync_copy(acc_sh.at[pl.ds(sid * RPT, RPT)], zrow_v)
    pltpu.sync_copy(zrow_v, out_hbm.at[pl.ds(cid * NP + sid * RPT, RPT)])


def _deg_call(dst):
    k = pl.kernel(
        _deg_body,
        out_type=jax.ShapeDtypeStruct((2 * NP,), jnp.float32),
        mesh=_mesh(),
        scratch_types=[
            pltpu.VMEM((CH,), jnp.int32),
            pltpu.VMEM((CH,), jnp.float32),
            pltpu.VMEM((RPT,), jnp.float32),
            pltpu.VMEM_SHARED((NP,), jnp.float32),
            pltpu.SemaphoreType.DMA,
        ],
    )
    return k(dst)


def _agg_body(src_hbm, dst_hbm, hs_hbm, out_hbm, src_v, dst_v, rows_v,
              acc_sh, sem):
    cid = lax.axis_index("c")
    sid = lax.axis_index("s")
    wid = sid * 2 + cid

    _zero_rows(rows_v, CH)
    for j in range(RPT // CH):
        pltpu.sync_copy(rows_v, acc_sh.at[pl.ds(sid * RPT + j * CH, CH)])
    plsc.subcore_barrier()

    lo, cnt = _chunk_range(wid)

    def chunk(c, _):
        base = (lo + c) * CH
        pltpu.sync_copy(src_hbm.at[pl.ds(base, CH)], src_v)
        pltpu.sync_copy(dst_hbm.at[pl.ds(base, CH)], dst_v)
        pltpu.async_copy(hs_hbm.at[src_v], rows_v, sem).wait()
        pltpu.sync_copy(rows_v, acc_sh.at[dst_v], add=True)
        return 0
    lax.fori_loop(0, cnt, chunk, 0)
    plsc.subcore_barrier()

    for j in range(RPT // CH):
        r0 = sid * RPT + j * CH
        pltpu.sync_copy(acc_sh.at[pl.ds(r0, CH)], rows_v)
        pltpu.sync_copy(rows_v, out_hbm.at[pl.ds(cid * NP + r0, CH)])


def _agg_call(src, dst, hs):
    k = pl.kernel(
        _agg_body,
        out_type=jax.ShapeDtypeStruct((2 * NP, D), jnp.float32),
        mesh=_mesh(),
        scratch_types=[
            pltpu.VMEM((CH,), jnp.int32),
            pltpu.VMEM((CH,), jnp.int32),
            pltpu.VMEM((CH, D), jnp.float32),
            pltpu.VMEM_SHARED((NP, D), jnp.float32),
            pltpu.SemaphoreType.DMA,
        ],
    )
    return k(src, dst, hs)


_DOT = dict(preferred_element_type=jnp.float32)


def _mm1_body(x_ref, w_ref, degp_ref, hs_ref, dinv_ref):
    i = pl.program_id(0)
    deg = (degp_ref[0, pl.ds(i * BLK, BLK)]
           + degp_ref[1, pl.ds(i * BLK, BLK)] + 1.0)
    dinv = lax.rsqrt(jnp.maximum(deg, 1.0))
    h = jnp.dot(x_ref[...], w_ref[...], **_DOT)
    hs_ref[...] = h * dinv[:, None]
    dinv_ref[...] = dinv[None, :]


def _mm1_call(xp, W, degp):
    return pl.pallas_call(
        _mm1_body,
        grid=(NBLK,),
        in_specs=[
            pl.BlockSpec((BLK, D), lambda i: (i, 0)),
            pl.BlockSpec((D, D), lambda i: (0, 0)),
            pl.BlockSpec((2, NP), lambda i: (0, 0)),
        ],
        out_specs=[
            pl.BlockSpec((BLK, D), lambda i: (i, 0)),
            pl.BlockSpec((1, BLK), lambda i: (0, i)),
        ],
        out_shape=[
            jax.ShapeDtypeStruct((NP, D), jnp.float32),
            jax.ShapeDtypeStruct((1, NP), jnp.float32),
        ],
    )(xp, W, degp)


def _bn_mm_body(pre_ref, cs_ref, cq_ref, g_ref, bt_ref, w_ref, dinv_ref,
                hs_ref):
    i = pl.program_id(0)
    mean = cs_ref[...] / N
    var = cq_ref[...] / N - mean * mean
    inv = lax.rsqrt(var + 1e-5)
    a = (pre_ref[...] - mean) * inv * g_ref[...] + bt_ref[...]
    act = jnp.where(a >= 0, a, 0.01 * a)
    h = jnp.dot(act, w_ref[...], **_DOT)
    dinv = dinv_ref[0, pl.ds(i * BLK, BLK)]
    hs_ref[...] = h * dinv[:, None]


def _bn_mm_call(pre, cs, cq, g, bt, W, dinv):
    return pl.pallas_call(
        _bn_mm_body,
        grid=(NBLK,),
        in_specs=[
            pl.BlockSpec((BLK, D), lambda i: (i, 0)),
            pl.BlockSpec((1, D), lambda i: (0, 0)),
            pl.BlockSpec((1, D), lambda i: (0, 0)),
            pl.BlockSpec((1, D), lambda i: (0, 0)),
            pl.BlockSpec((1, D), lambda i: (0, 0)),
            pl.BlockSpec((D, D), lambda i: (0, 0)),
            pl.BlockSpec((1, NP), lambda i: (0, 0)),
        ],
        out_specs=pl.BlockSpec((BLK, D), lambda i: (i, 0)),
        out_shape=jax.ShapeDtypeStruct((NP, D), jnp.float32),
    )(pre, cs, cq, g, bt, W, dinv)


def _comb_body(a0_ref, a1_ref, hs_ref, b_ref, dinv_ref, pre_ref,
               cs_ref, cq_ref):
    i = pl.program_id(0)
    dinv = dinv_ref[0, pl.ds(i * BLK, BLK)]
    pre = dinv[:, None] * (a0_ref[...] + a1_ref[...] + hs_ref[...]) + b_ref[...]
    pre_ref[...] = pre

    rows = i * BLK + lax.broadcasted_iota(jnp.int32, (BLK, 1), 0)
    masked = jnp.where(rows < N, pre, 0.0)

    @pl.when(i == 0)
    def _():
        cs_ref[...] = jnp.zeros((1, D), jnp.float32)
        cq_ref[...] = jnp.zeros((1, D), jnp.float32)

    cs_ref[...] += jnp.sum(masked, axis=0, keepdims=True)
    cq_ref[...] += jnp.sum(masked * masked, axis=0, keepdims=True)


def _comb_call(aggp, hs, b, dinv):
    return pl.pallas_call(
        _comb_body,
        grid=(NBLK,),
        in_specs=[
            pl.BlockSpec((BLK, D), lambda i: (i, 0)),
            pl.BlockSpec((BLK, D), lambda i: (i + NBLK, 0)),
            pl.BlockSpec((BLK, D), lambda i: (i, 0)),
            pl.BlockSpec((1, D), lambda i: (0, 0)),
            pl.BlockSpec((1, NP), lambda i: (0, 0)),
        ],
        out_specs=[
            pl.BlockSpec((BLK, D), lambda i: (i, 0)),
            pl.BlockSpec((1, D), lambda i: (0, 0)),
            pl.BlockSpec((1, D), lambda i: (0, 0)),
        ],
        out_shape=[
            jax.ShapeDtypeStruct((NP, D), jnp.float32),
            jax.ShapeDtypeStruct((1, D), jnp.float32),
            jax.ShapeDtypeStruct((1, D), jnp.float32),
        ],
    )(aggp, aggp, hs, b, dinv)


def _pool_body(pre_ref, cs_ref, cq_ref, g_ref, bt_ref, batch_ref, lw_ref,
               lb_ref, sums_ref, cnts_ref, out_ref):
    i = pl.program_id(0)
    mean = cs_ref[...] / N
    var = cq_ref[...] / N - mean * mean
    inv = lax.rsqrt(var + 1e-5)
    a = (pre_ref[...] - mean) * inv * g_ref[...] + bt_ref[...]
    act = jnp.where(a >= 0, a, 0.01 * a)

    bids = batch_ref[0, pl.ds(i * BLK, BLK)]
    rows = i * BLK + lax.broadcasted_iota(jnp.int32, (BLK,), 0)
    gid = lax.broadcasted_iota(jnp.int32, (G, BLK), 0)
    onehot = jnp.where((bids[None, :] == gid) & (rows[None, :] < N),
                       1.0, 0.0)

    @pl.when(i == 0)
    def _():
        sums_ref[...] = jnp.zeros((G, D), jnp.float32)
        cnts_ref[...] = jnp.zeros((G, D), jnp.float32)

    sums_ref[...] += jnp.dot(onehot, act, **_DOT)
    cnts_ref[...] += jnp.sum(onehot, axis=1, keepdims=True)

    @pl.when(i == NBLK - 1)
    def _():
        pooled = sums_ref[...] / jnp.maximum(cnts_ref[...], 1.0)
        r = jnp.dot(pooled, lw_ref[...], **_DOT) + lb_ref[...]
        out_ref[...] = jnp.broadcast_to(r, (G, D))


def _pool_call(pre, cs, cq, g, bt, batch_p, lw, lb):
    return pl.pallas_call(
        _pool_body,
        grid=(NBLK,),
        in_specs=[
            pl.BlockSpec((BLK, D), lambda i: (i, 0)),
            pl.BlockSpec((1, D), lambda i: (0, 0)),
            pl.BlockSpec((1, D), lambda i: (0, 0)),
            pl.BlockSpec((1, D), lambda i: (0, 0)),
            pl.BlockSpec((1, D), lambda i: (0, 0)),
            pl.BlockSpec((1, NP), lambda i: (0, 0)),
            pl.BlockSpec((D, 1), lambda i: (0, 0)),
            pl.BlockSpec((1, 1), lambda i: (0, 0)),
        ],
        out_specs=[
            pl.BlockSpec((G, D), lambda i: (0, 0)),
            pl.BlockSpec((G, D), lambda i: (0, 0)),
            pl.BlockSpec((G, D), lambda i: (0, 0)),
        ],
        out_shape=[
            jax.ShapeDtypeStruct((G, D), jnp.float32),
            jax.ShapeDtypeStruct((G, D), jnp.float32),
            jax.ShapeDtypeStruct((G, D), jnp.float32),
        ],
    )(pre, cs, cq, g, bt, batch_p, lw, lb)


def kernel(x, edge_index, batch, W1, b1, g1, bt1, W2, b2, g2, bt2,
           W3, b3, g3, bt3, lw, lb):
    src = edge_index[0]
    dst = edge_index[1]
    xp = jnp.pad(x, ((0, NP - N), (0, 0)))
    batch_p = jnp.pad(batch, (0, NP - N), constant_values=G).reshape(1, NP)

    degp = _deg_call(dst).reshape(2, NP)
    hs, dinv = _mm1_call(xp, W1, degp)

    layers = ((b1, g1, bt1, W2), (b2, g2, bt2, W3), (b3, g3, bt3, None))
    pre = cs = cq = None
    for (b, g, bt, Wn) in layers:
        aggp = _agg_call(src, dst, hs)
        pre, cs, cq = _comb_call(aggp, hs, b.reshape(1, D), dinv)
        if Wn is not None:
            hs = _bn_mm_call(pre, cs, cq, g.reshape(1, D), bt.reshape(1, D),
                             Wn, dinv)

    out = _pool_call(pre, cs, cq, g3.reshape(1, D), bt3.reshape(1, D),
                     batch_p, lw, lb.reshape(1, 1))[2]
    return out[:, 0:1]


# speedup vs baseline: 19.9097x; 1.4907x over previous
import functools

import jax
import jax.numpy as jnp
from jax import lax
from jax.experimental import pallas as pl
from jax.experimental.pallas import tpu as pltpu
from jax.experimental.pallas import tpu_sc as plsc

N = 10000
NP = 10240
E = 320000
D = 128
G = 16
BLK = 1024
NBLK = NP // BLK
CH = 128
NCH = E // CH
NW = 32
RPT = NP // 16

@functools.lru_cache(maxsize=1)
def _mesh():
    return plsc.VectorSubcoreMesh(core_axis_name="c", subcore_axis_name="s")


def _chunk_range(wid):
    base = NCH // NW
    rem = NCH % NW
    lo = wid * base + jnp.minimum(wid, rem)
    cnt = base + jnp.where(wid < rem, 1, 0)
    return lo, cnt


def _zero_rows(rows_v, nrows):
    def body(r, _):
        for j in range(8):
            rows_v[r, pl.ds(j * 16, 16)] = jnp.zeros((16,), jnp.float32)
        return 0
    lax.fori_loop(0, nrows, body, 0)


def _deg_body(dst_hbm, out_hbm, dst_v, ones_v, zrow_v, acc_sh, sem):
    cid = lax.axis_index("c")
    sid = lax.axis_index("s")
    wid = sid * 2 + cid

    for j in range(8):
        ones_v[pl.ds(j * 16, 16)] = jnp.ones((16,), jnp.float32)
    def zb(i, _):
        zrow_v[pl.ds(i * 16, 16)] = jnp.zeros((16,), jnp.float32)
        return 0
    lax.fori_loop(0, RPT // 16, zb, 0)
    pltpu.sync_copy(zrow_v, acc_sh.at[pl.ds(sid * RPT, RPT)])
    plsc.subcore_barrier()

    lo, cnt = _chunk_range(wid)

    def chunk(c, _):
        base = (lo + c) * CH
        pltpu.sync_copy(dst_hbm.at[pl.ds(base, CH)], dst_v)
        pltpu.sync_copy(ones_v, acc_sh.at[dst_v], add=True)
        return 0
    lax.fori_loop(0, cnt, chunk, 0)
    plsc.subcore_barrier()

    pltpu.sync_copy(acc_sh.at[pl.ds(sid * RPT, RPT)], zrow_v)
    pltpu.sync_copy(zrow_v, out_hbm.at[pl.ds(cid * NP + sid * RPT, RPT)])


def _deg_call(dst):
    k = pl.kernel(
        _deg_body,
        out_type=jax.ShapeDtypeStruct((2 * NP,), jnp.float32),
        mesh=_mesh(),
        scratch_types=[
            pltpu.VMEM((CH,), jnp.int32),
            pltpu.VMEM((CH,), jnp.float32),
            pltpu.VMEM((RPT,), jnp.float32),
            pltpu.VMEM_SHARED((NP,), jnp.float32),
            pltpu.SemaphoreType.DMA,
        ],
    )
    return k(dst)


def _agg_body(src_hbm, dst_hbm, hs_hbm, out_hbm, src0, dst0, rows0,
              src1, dst1, rows1, acc_sh, sem0, sem1):
    cid = lax.axis_index("c")
    sid = lax.axis_index("s")
    wid = sid * 2 + cid

    _zero_rows(rows0, CH)
    for j in range(RPT // CH):
        pltpu.sync_copy(rows0, acc_sh.at[pl.ds(sid * RPT + j * CH, CH)])
    plsc.subcore_barrier()

    lo, cnt = _chunk_range(wid)

    bufs = ((src0, dst0, rows0, sem0), (src1, dst1, rows1, sem1))

    def _issue(c, sv, dv, rv, sem):
        base = (lo + c) * CH
        pltpu.sync_copy(src_hbm.at[pl.ds(base, CH)], sv)
        pltpu.sync_copy(dst_hbm.at[pl.ds(base, CH)], dv)
        pltpu.async_copy(hs_hbm.at[sv], rv, sem)

    for k in range(2):
        sv, dv, rv, sem = bufs[k]

        @pl.when(cnt > k)
        def _(k=k, sv=sv, dv=dv, rv=rv, sem=sem):
            _issue(k, sv, dv, rv, sem)

    def pair(p, _):
        for k in range(2):
            c = 2 * p + k
            sv, dv, rv, sem = bufs[k]

            @pl.when(c < cnt)
            def _(c=c, sv=sv, dv=dv, rv=rv, sem=sem):
                pltpu.make_async_copy(hs_hbm.at[sv], rv, sem).wait()
                pltpu.sync_copy(rv, acc_sh.at[dv], add=True)

                @pl.when(c + 2 < cnt)
                def _():
                    _issue(c + 2, sv, dv, rv, sem)
        return 0
    lax.fori_loop(0, (cnt + 1) // 2, pair, 0)
    plsc.subcore_barrier()

    for j in range(RPT // CH):
        r0 = sid * RPT + j * CH
        pltpu.sync_copy(acc_sh.at[pl.ds(r0, CH)], rows0)
        pltpu.sync_copy(rows0, out_hbm.at[pl.ds(cid * NP + r0, CH)])


def _agg_call(src, dst, hs):
    k = pl.kernel(
        _agg_body,
        out_type=jax.ShapeDtypeStruct((2 * NP, D), jnp.float32),
        mesh=_mesh(),
        scratch_types=[
            pltpu.VMEM((CH,), jnp.int32),
            pltpu.VMEM((CH,), jnp.int32),
            pltpu.VMEM((CH, D), jnp.float32),
            pltpu.VMEM((CH,), jnp.int32),
            pltpu.VMEM((CH,), jnp.int32),
            pltpu.VMEM((CH, D), jnp.float32),
            pltpu.VMEM_SHARED((NP, D), jnp.float32),
            pltpu.SemaphoreType.DMA,
            pltpu.SemaphoreType.DMA,
        ],
    )
    return k(src, dst, hs)


_DOT = dict(preferred_element_type=jnp.float32)


def _mm1_body(x_ref, w_ref, degp_ref, hs_ref, dinv_ref):
    i = pl.program_id(0)
    deg = (degp_ref[0, pl.ds(i * BLK, BLK)]
           + degp_ref[1, pl.ds(i * BLK, BLK)] + 1.0)
    dinv = lax.rsqrt(jnp.maximum(deg, 1.0))
    h = jnp.dot(x_ref[...], w_ref[...], **_DOT)
    hs_ref[...] = h * dinv[:, None]
    dinv_ref[...] = dinv[None, :]


def _mm1_call(xp, W, degp):
    return pl.pallas_call(
        _mm1_body,
        grid=(NBLK,),
        in_specs=[
            pl.BlockSpec((BLK, D), lambda i: (i, 0)),
            pl.BlockSpec((D, D), lambda i: (0, 0)),
            pl.BlockSpec((2, NP), lambda i: (0, 0)),
        ],
        out_specs=[
            pl.BlockSpec((BLK, D), lambda i: (i, 0)),
            pl.BlockSpec((1, BLK), lambda i: (0, i)),
        ],
        out_shape=[
            jax.ShapeDtypeStruct((NP, D), jnp.float32),
            jax.ShapeDtypeStruct((1, NP), jnp.float32),
        ],
    )(xp, W, degp)


def _bn_mm_body(pre_ref, cs_ref, cq_ref, g_ref, bt_ref, w_ref, dinv_ref,
                hs_ref):
    i = pl.program_id(0)
    mean = cs_ref[...] / N
    var = cq_ref[...] / N - mean * mean
    inv = lax.rsqrt(var + 1e-5)
    a = (pre_ref[...] - mean) * inv * g_ref[...] + bt_ref[...]
    act = jnp.where(a >= 0, a, 0.01 * a)
    h = jnp.dot(act, w_ref[...], **_DOT)
    dinv = dinv_ref[0, pl.ds(i * BLK, BLK)]
    hs_ref[...] = h * dinv[:, None]


def _bn_mm_call(pre, cs, cq, g, bt, W, dinv):
    return pl.pallas_call(
        _bn_mm_body,
        grid=(NBLK,),
        in_specs=[
            pl.BlockSpec((BLK, D), lambda i: (i, 0)),
            pl.BlockSpec((1, D), lambda i: (0, 0)),
            pl.BlockSpec((1, D), lambda i: (0, 0)),
            pl.BlockSpec((1, D), lambda i: (0, 0)),
            pl.BlockSpec((1, D), lambda i: (0, 0)),
            pl.BlockSpec((D, D), lambda i: (0, 0)),
            pl.BlockSpec((1, NP), lambda i: (0, 0)),
        ],
        out_specs=pl.BlockSpec((BLK, D), lambda i: (i, 0)),
        out_shape=jax.ShapeDtypeStruct((NP, D), jnp.float32),
    )(pre, cs, cq, g, bt, W, dinv)


def _comb_body(a0_ref, a1_ref, hs_ref, b_ref, dinv_ref, pre_ref,
               cs_ref, cq_ref):
    i = pl.program_id(0)
    dinv = dinv_ref[0, pl.ds(i * BLK, BLK)]
    pre = dinv[:, None] * (a0_ref[...] + a1_ref[...] + hs_ref[...]) + b_ref[...]
    pre_ref[...] = pre

    rows = i * BLK + lax.broadcasted_iota(jnp.int32, (BLK, 1), 0)
    masked = jnp.where(rows < N, pre, 0.0)

    @pl.when(i == 0)
    def _():
        cs_ref[...] = jnp.zeros((1, D), jnp.float32)
        cq_ref[...] = jnp.zeros((1, D), jnp.float32)

    cs_ref[...] += jnp.sum(masked, axis=0, keepdims=True)
    cq_ref[...] += jnp.sum(masked * masked, axis=0, keepdims=True)


def _comb_call(aggp, hs, b, dinv):
    return pl.pallas_call(
        _comb_body,
        grid=(NBLK,),
        in_specs=[
            pl.BlockSpec((BLK, D), lambda i: (i, 0)),
            pl.BlockSpec((BLK, D), lambda i: (i + NBLK, 0)),
            pl.BlockSpec((BLK, D), lambda i: (i, 0)),
            pl.BlockSpec((1, D), lambda i: (0, 0)),
            pl.BlockSpec((1, NP), lambda i: (0, 0)),
        ],
        out_specs=[
            pl.BlockSpec((BLK, D), lambda i: (i, 0)),
            pl.BlockSpec((1, D), lambda i: (0, 0)),
            pl.BlockSpec((1, D), lambda i: (0, 0)),
        ],
        out_shape=[
            jax.ShapeDtypeStruct((NP, D), jnp.float32),
            jax.ShapeDtypeStruct((1, D), jnp.float32),
            jax.ShapeDtypeStruct((1, D), jnp.float32),
        ],
    )(aggp, aggp, hs, b, dinv)


def _pool_body(pre_ref, cs_ref, cq_ref, g_ref, bt_ref, batch_ref, lw_ref,
               lb_ref, sums_ref, cnts_ref, out_ref):
    i = pl.program_id(0)
    mean = cs_ref[...] / N
    var = cq_ref[...] / N - mean * mean
    inv = lax.rsqrt(var + 1e-5)
    a = (pre_ref[...] - mean) * inv * g_ref[...] + bt_ref[...]
    act = jnp.where(a >= 0, a, 0.01 * a)

    bids = batch_ref[0, pl.ds(i * BLK, BLK)]
    rows = i * BLK + lax.broadcasted_iota(jnp.int32, (BLK,), 0)
    gid = lax.broadcasted_iota(jnp.int32, (G, BLK), 0)
    onehot = jnp.where((bids[None, :] == gid) & (rows[None, :] < N),
                       1.0, 0.0)

    @pl.when(i == 0)
    def _():
        sums_ref[...] = jnp.zeros((G, D), jnp.float32)
        cnts_ref[...] = jnp.zeros((G, D), jnp.float32)

    sums_ref[...] += jnp.dot(onehot, act, **_DOT)
    cnts_ref[...] += jnp.sum(onehot, axis=1, keepdims=True)

    @pl.when(i == NBLK - 1)
    def _():
        pooled = sums_ref[...] / jnp.maximum(cnts_ref[...], 1.0)
        r = jnp.dot(pooled, lw_ref[...], **_DOT) + lb_ref[...]
        out_ref[...] = jnp.broadcast_to(r, (G, D))


def _pool_call(pre, cs, cq, g, bt, batch_p, lw, lb):
    return pl.pallas_call(
        _pool_body,
        grid=(NBLK,),
        in_specs=[
            pl.BlockSpec((BLK, D), lambda i: (i, 0)),
            pl.BlockSpec((1, D), lambda i: (0, 0)),
            pl.BlockSpec((1, D), lambda i: (0, 0)),
            pl.BlockSpec((1, D), lambda i: (0, 0)),
            pl.BlockSpec((1, D), lambda i: (0, 0)),
            pl.BlockSpec((1, NP), lambda i: (0, 0)),
            pl.BlockSpec((D, 1), lambda i: (0, 0)),
            pl.BlockSpec((1, 1), lambda i: (0, 0)),
        ],
        out_specs=[
            pl.BlockSpec((G, D), lambda i: (0, 0)),
            pl.BlockSpec((G, D), lambda i: (0, 0)),
            pl.BlockSpec((G, D), lambda i: (0, 0)),
        ],
        out_shape=[
            jax.ShapeDtypeStruct((G, D), jnp.float32),
            jax.ShapeDtypeStruct((G, D), jnp.float32),
            jax.ShapeDtypeStruct((G, D), jnp.float32),
        ],
    )(pre, cs, cq, g, bt, batch_p, lw, lb)


def kernel(x, edge_index, batch, W1, b1, g1, bt1, W2, b2, g2, bt2,
           W3, b3, g3, bt3, lw, lb):
    src = edge_index[0]
    dst = edge_index[1]
    xp = jnp.pad(x, ((0, NP - N), (0, 0)))
    batch_p = jnp.pad(batch, (0, NP - N), constant_values=G).reshape(1, NP)

    degp = _deg_call(dst).reshape(2, NP)
    hs, dinv = _mm1_call(xp, W1, degp)

    layers = ((b1, g1, bt1, W2), (b2, g2, bt2, W3), (b3, g3, bt3, None))
    pre = cs = cq = None
    for (b, g, bt, Wn) in layers:
        aggp = _agg_call(src, dst, hs)
        pre, cs, cq = _comb_call(aggp, hs, b.reshape(1, D), dinv)
        if Wn is not None:
            hs = _bn_mm_call(pre, cs, cq, g.reshape(1, D), bt.reshape(1, D),
                             Wn, dinv)

    out = _pool_call(pre, cs, cq, g3.reshape(1, D), bt3.reshape(1, D),
                     batch_p, lw, lb.reshape(1, 1))[2]
    return out[:, 0:1]
